# baseline (speedup 1.0000x reference)
import functools

import jax
import jax.numpy as jnp
from jax import lax
from jax.experimental import pallas as pl
from jax.experimental.pallas import tpu as pltpu
from jax.experimental.pallas import tpu_sc as plsc

VOCABP = 1024
EMB = 256
NKEY = 2048
NCAND = 1000
NPER = 128
TOK = 64
SENTINEL = 1000
EPS = 1e-6

BLK = 512
R_KEYS = 0
R_VALS = NKEY
R_PERS = 2 * NKEY
R_XS = 2 * NKEY + NPER
R_CAND = 4608
ROWSP = R_CAND + 1024
NBLK = ROWSP // BLK


GROUPS = ROWSP // 16


def _build_counts_sc(idx):
    nc, ns = 2, 16
    nw = nc * ns
    gpt = (GROUPS + nw - 1) // nw
    mesh = plsc.VectorSubcoreMesh(core_axis_name="c", subcore_axis_name="s")

    @functools.partial(
        pl.kernel,
        mesh=mesh,
        compiler_params=pltpu.CompilerParams(needs_layout_passes=False),
        out_type=jax.ShapeDtypeStruct((ROWSP, VOCABP), jnp.float32),
        scratch_types=[
            pltpu.VMEM((4, 16, TOK), jnp.int32),
            pltpu.VMEM((2, 16, VOCABP), jnp.float32),
            pltpu.SemaphoreType.DMA,
            pltpu.SemaphoreType.DMA,
            pltpu.SemaphoreType.DMA,
            pltpu.SemaphoreType.DMA,
            pltpu.SemaphoreType.DMA,
            pltpu.SemaphoreType.DMA,
        ],
    )
    def k(idx_hbm, out_hbm, idx_v, cnt_v, os0, os1, is0, is1, is2, is3):
        wid = lax.axis_index("s") * nc + lax.axis_index("c")
        iota16 = lax.iota(jnp.int32, 16)
        ones = jnp.ones((16,), jnp.float32)
        zeros = jnp.zeros((16,), jnp.float32)
        osems = (os0, os1)
        isems = (is0, is1, is2, is3)

        def idx_fetch(g, s, start):
            @pl.when(g < GROUPS)
            def _():
                cp = (pltpu.async_copy if start else pltpu.make_async_copy)(
                    idx_hbm.at[pl.ds(g * 16, 16)], idx_v.at[s], isems[s])
                if not start:
                    cp.wait()

        idx_fetch(wid, 0, True)

        def zbody(j, _):
            for b in range(2):
                for r in range(16):
                    cnt_v[b, r, pl.ds(j * 16, 16)] = zeros
            return 0
        lax.fori_loop(0, VOCABP // 16, zbody, 0)

        def scatter_pass(s, b, val):
            def tbody(t, _):
                tvec = jnp.full((16,), t, jnp.int32)
                tok = plsc.load_gather(idx_v.at[s], [iota16, tvec])
                plsc.addupdate_scatter(cnt_v.at[b], [iota16, tok], val)
                return 0
            lax.fori_loop(0, TOK, tbody, 0, unroll=8)

        nblk = -(-gpt // 4)

        def block(i, _):
            for u in range(4):
                jj = i * 4 + u
                g = jj * nw + wid
                b = u % 2
                idx_fetch(g + nw, (u + 1) % 4, True)

                @pl.when((g < GROUPS) & (jj >= 2))
                def _():
                    gprev = g - 2 * nw
                    pltpu.make_async_copy(
                        cnt_v.at[b], out_hbm.at[pl.ds(gprev * 16, 16)], osems[b]
                    ).wait()
                    scatter_pass((u + 2) % 4, b, -ones)
                idx_fetch(g, u, False)

                @pl.when(g < GROUPS)
                def _():
                    scatter_pass(u, b, ones)
                    pltpu.async_copy(
                        cnt_v.at[b], out_hbm.at[pl.ds(g * 16, 16)], osems[b]
                    )
            return 0

        lax.fori_loop(0, nblk, block, 0)

        for b in range(2):
            pltpu.make_async_copy(
                cnt_v.at[b], out_hbm.at[pl.ds(0, 16)], osems[b]
            ).wait()

    return k(idx)


def _softmax_row(x):
    m = jnp.max(x, axis=1, keepdims=True)
    e = jnp.exp(x - m)
    return e / jnp.sum(e, axis=1, keepdims=True)


def _row_norms(enc):
    n = jnp.sqrt(jnp.sum(enc * enc, axis=1))
    return jnp.maximum(n, EPS)[None, :]


def _vnorm(q):
    return jnp.maximum(jnp.sqrt(jnp.sum(q * q)), EPS)


def _dense_body(cnt_ref, tab_ref, r_ref, r2_ref, out_ref, enc_ref):
    f32 = jnp.float32
    i = pl.program_id(0)
    enc_ref[pl.ds(i * BLK, BLK), :] = jnp.dot(
        cnt_ref[...], tab_ref[0], preferred_element_type=f32)

    @pl.when(i == NBLK - 1)
    def _():
        enc_k = enc_ref[R_KEYS:R_KEYS + NKEY, :]
        enc_v = enc_ref[R_VALS:R_VALS + NKEY, :]
        enc_p = enc_ref[R_PERS:R_PERS + NPER, :]
        enc_c = enc_ref[R_CAND:R_CAND + NCAND, :]
        q = enc_ref[R_XS:R_XS + 1, :]

        nk = _row_norms(enc_k)
        nc = _row_norms(enc_c)
        np_ = _row_norms(enc_p)

        def dotq(v, enc):
            return lax.dot_general(v, enc, (((1,), (1,)), ((), ())),
                                   preferred_element_type=f32)

        for rm_ref in (r_ref, r2_ref):
            Rm = rm_ref[...]
            cos = dotq(q, enc_p) / (np_ * _vnorm(q))
            ret = _softmax_row(cos)
            hop = jnp.dot(ret, enc_p, preferred_element_type=f32)
            q_plus = dotq(q + hop, Rm)
            cos2 = dotq(q_plus, enc_k) / (nk * _vnorm(q_plus))
            ret2 = _softmax_row(cos2)
            hop2 = jnp.dot(ret2, enc_v, preferred_element_type=f32)
            q = dotq(q_plus + hop2, Rm)

        out_ref[...] = dotq(q, enc_c) / (nc * _vnorm(q))


def _dense(counts, tables, R, R2):
    return pl.pallas_call(
        _dense_body,
        grid=(NBLK,),
        in_specs=[
            pl.BlockSpec((BLK, VOCABP), lambda i: (i, 0)),
            pl.BlockSpec((1, VOCABP, EMB),
                         lambda i: (jnp.where(i >= R_CAND // BLK, 1, 0), 0, 0)),
            pl.BlockSpec((EMB, EMB), lambda i: (0, 0)),
            pl.BlockSpec((EMB, EMB), lambda i: (0, 0)),
        ],
        out_specs=pl.BlockSpec((1, NCAND), lambda i: (0, 0)),
        out_shape=jax.ShapeDtypeStruct((1, NCAND), jnp.float32),
        scratch_shapes=[pltpu.VMEM((ROWSP, EMB), jnp.float32)],
    )(counts, tables, R, R2)


def kernel(xs, candidates, persona, keys, values, label, shared_emb, cand_emb, R, R2):
    del label
    i32 = jnp.int32
    xs_pad = jnp.pad(xs.astype(i32), ((0, 0), (0, TOK - xs.shape[1])),
                     constant_values=SENTINEL)
    shared_rows = jnp.concatenate([
        keys.astype(i32), values.astype(i32), persona.astype(i32), xs_pad,
    ], axis=0)
    shared_rows = jnp.pad(shared_rows, ((0, R_CAND - (2 * NKEY + NPER + 1)), (0, 0)),
                          constant_values=SENTINEL)
    cand_rows = jnp.pad(candidates.astype(i32),
                        ((0, ROWSP - R_CAND - NCAND), (0, 0)),
                        constant_values=SENTINEL)
    idx = jnp.concatenate([shared_rows, cand_rows], axis=0)

    epad = jnp.pad(shared_emb, ((0, VOCABP - shared_emb.shape[0]), (0, 0)))
    cepad = jnp.pad(cand_emb, ((0, VOCABP - cand_emb.shape[0]), (0, 0)))
    tables = jnp.stack([epad, cepad], axis=0)

    counts = _build_counts_sc(idx)
    preds = _dense(counts, tables, R, R2)
    return preds.reshape(NCAND)


# speedup vs baseline: 1.0505x; 1.0505x over previous
import functools

import jax
import jax.numpy as jnp
from jax import lax
from jax.experimental import pallas as pl
from jax.experimental.pallas import tpu as pltpu
from jax.experimental.pallas import tpu_sc as plsc

VOCABP = 1024
EMB = 256
NKEY = 2048
NCAND = 1000
NPER = 128
TOK = 64
SENTINEL = 1000
EPS = 1e-6

BLK = 512
R_KEYS = 0
R_VALS = NKEY
R_PERS = 2 * NKEY
R_XS = 2 * NKEY + NPER
R_CAND = 4608
ROWSP = R_CAND + 1024
NBLK = ROWSP // BLK


GROUPS = ROWSP // 16


def _build_counts_sc(idx):
    nc, ns = 2, 16
    nw = nc * ns
    gpt = (GROUPS + nw - 1) // nw
    mesh = plsc.VectorSubcoreMesh(core_axis_name="c", subcore_axis_name="s")

    @functools.partial(
        pl.kernel,
        mesh=mesh,
        compiler_params=pltpu.CompilerParams(needs_layout_passes=False),
        out_type=jax.ShapeDtypeStruct((ROWSP, VOCABP), jnp.float32),
        scratch_types=[
            pltpu.VMEM((4, 16, TOK), jnp.int32),
            pltpu.VMEM((2, 16, VOCABP), jnp.float32),
            pltpu.SemaphoreType.DMA,
            pltpu.SemaphoreType.DMA,
            pltpu.SemaphoreType.DMA,
            pltpu.SemaphoreType.DMA,
            pltpu.SemaphoreType.DMA,
            pltpu.SemaphoreType.DMA,
        ],
    )
    def k(idx_hbm, out_hbm, idx_v, cnt_v, os0, os1, is0, is1, is2, is3):
        wid = lax.axis_index("s") * nc + lax.axis_index("c")
        iota16 = lax.iota(jnp.int32, 16)
        ones = jnp.ones((16,), jnp.float32)
        zeros = jnp.zeros((16,), jnp.float32)
        osems = (os0, os1)
        isems = (is0, is1, is2, is3)

        def idx_fetch(g, s, start):
            @pl.when(g < GROUPS)
            def _():
                cp = (pltpu.async_copy if start else pltpu.make_async_copy)(
                    idx_hbm.at[pl.ds(g * 16, 16)], idx_v.at[s], isems[s])
                if not start:
                    cp.wait()

        idx_fetch(wid, 0, True)

        def zbody(j, _):
            for b in range(2):
                for r in range(16):
                    cnt_v[b, r, pl.ds(j * 16, 16)] = zeros
            return 0
        lax.fori_loop(0, VOCABP // 16, zbody, 0)

        def scatter_pass(s, b, val):
            def tbody(t, _):
                tvec = jnp.full((16,), t, jnp.int32)
                tok = plsc.load_gather(idx_v.at[s], [iota16, tvec])
                plsc.addupdate_scatter(cnt_v.at[b], [iota16, tok], val)
                return 0
            lax.fori_loop(0, TOK, tbody, 0, unroll=8)

        nblk = -(-gpt // 4)

        def block(i, _):
            for u in range(4):
                jj = i * 4 + u
                g = jj * nw + wid
                b = u % 2
                idx_fetch(g + nw, (u + 1) % 4, True)

                @pl.when((g < GROUPS) & (jj >= 2))
                def _():
                    gprev = g - 2 * nw
                    pltpu.make_async_copy(
                        cnt_v.at[b], out_hbm.at[pl.ds(gprev * 16, 16)], osems[b]
                    ).wait()
                    scatter_pass((u + 2) % 4, b, -ones)
                idx_fetch(g, u, False)

                @pl.when(g < GROUPS)
                def _():
                    scatter_pass(u, b, ones)
                    pltpu.async_copy(
                        cnt_v.at[b], out_hbm.at[pl.ds(g * 16, 16)], osems[b]
                    )
            return 0

        lax.fori_loop(0, nblk, block, 0)

        for b in range(2):
            pltpu.make_async_copy(
                cnt_v.at[b], out_hbm.at[pl.ds(0, 16)], osems[b]
            ).wait()

    return k(idx)


def _softmax_row(x):
    m = jnp.max(x, axis=1, keepdims=True)
    e = jnp.exp(x - m)
    return e / jnp.sum(e, axis=1, keepdims=True)


def _row_norms(enc):
    n = jnp.sqrt(jnp.sum(enc * enc, axis=1))
    return jnp.maximum(n, EPS)[None, :]


def _vnorm(q):
    return jnp.maximum(jnp.sqrt(jnp.sum(q * q)), EPS)


def _dense_body(cnt_ref, e_ref, ce_ref, r_ref, r2_ref, out_ref):
    f32 = jnp.float32
    E = e_ref[...]
    enc_k = jnp.dot(cnt_ref[R_KEYS:R_KEYS + NKEY, :], E, preferred_element_type=f32)
    enc_v = jnp.dot(cnt_ref[R_VALS:R_VALS + NKEY, :], E, preferred_element_type=f32)
    enc_p = jnp.dot(cnt_ref[R_PERS:R_PERS + NPER, :], E, preferred_element_type=f32)
    q = jnp.dot(cnt_ref[R_XS:R_XS + 1, :], E, preferred_element_type=f32)
    enc_c = jnp.dot(cnt_ref[R_CAND:R_CAND + NCAND, :], ce_ref[...],
                    preferred_element_type=f32)

    nk = _row_norms(enc_k)
    nc = _row_norms(enc_c)
    np_ = _row_norms(enc_p)

    def dotq(v, enc):
        return lax.dot_general(v, enc, (((1,), (1,)), ((), ())),
                               preferred_element_type=f32)

    for rm_ref in (r_ref, r2_ref):
        Rm = rm_ref[...]
        cos = dotq(q, enc_p) / (np_ * _vnorm(q))
        ret = _softmax_row(cos)
        hop = jnp.dot(ret, enc_p, preferred_element_type=f32)
        q_plus = dotq(q + hop, Rm)
        cos2 = dotq(q_plus, enc_k) / (nk * _vnorm(q_plus))
        ret2 = _softmax_row(cos2)
        hop2 = jnp.dot(ret2, enc_v, preferred_element_type=f32)
        q = dotq(q_plus + hop2, Rm)

    out_ref[...] = dotq(q, enc_c) / (nc * _vnorm(q))


def _dense(counts, epad, cepad, R, R2):
    return pl.pallas_call(
        _dense_body,
        out_shape=jax.ShapeDtypeStruct((1, NCAND), jnp.float32),
    )(counts, epad, cepad, R, R2)


def kernel(xs, candidates, persona, keys, values, label, shared_emb, cand_emb, R, R2):
    del label
    i32 = jnp.int32
    xs_pad = jnp.pad(xs.astype(i32), ((0, 0), (0, TOK - xs.shape[1])),
                     constant_values=SENTINEL)
    shared_rows = jnp.concatenate([
        keys.astype(i32), values.astype(i32), persona.astype(i32), xs_pad,
    ], axis=0)
    shared_rows = jnp.pad(shared_rows, ((0, R_CAND - (2 * NKEY + NPER + 1)), (0, 0)),
                          constant_values=SENTINEL)
    cand_rows = jnp.pad(candidates.astype(i32),
                        ((0, ROWSP - R_CAND - NCAND), (0, 0)),
                        constant_values=SENTINEL)
    idx = jnp.concatenate([shared_rows, cand_rows], axis=0)

    epad = jnp.pad(shared_emb, ((0, VOCABP - shared_emb.shape[0]), (0, 0)))
    cepad = jnp.pad(cand_emb, ((0, VOCABP - cand_emb.shape[0]), (0, 0)))

    counts = _build_counts_sc(idx)
    preds = _dense(counts, epad, cepad, R, R2)
    return preds.reshape(NCAND)


# speedup vs baseline: 1.1787x; 1.1220x over previous
import functools

import jax
import jax.numpy as jnp
from jax import lax
from jax.experimental import pallas as pl
from jax.experimental.pallas import tpu as pltpu
from jax.experimental.pallas import tpu_sc as plsc

VOCABP = 1024
EMB = 256
NKEY = 2048
NCAND = 1000
NPER = 128
ROWS = NKEY + NKEY + NCAND + NPER + 1
ROWSP = 5232
TOK = 64
SENTINEL = 1000
EPS = 1e-6

R_KEYS = 0
R_VALS = NKEY
R_CAND = 2 * NKEY
R_PERS = 2 * NKEY + NCAND
R_XS = 2 * NKEY + NCAND + NPER


GROUPS = ROWSP // 16


def _build_counts_sc(idx):
    nc, ns = 2, 16
    nw = nc * ns
    gpt = (GROUPS + nw - 1) // nw
    mesh = plsc.VectorSubcoreMesh(core_axis_name="c", subcore_axis_name="s")

    @functools.partial(
        pl.kernel,
        mesh=mesh,
        compiler_params=pltpu.CompilerParams(needs_layout_passes=False),
        out_type=jax.ShapeDtypeStruct((ROWSP, VOCABP), jnp.float32),
        scratch_types=[
            pltpu.VMEM((4, 16, TOK), jnp.int32),
            pltpu.VMEM((2, 16, VOCABP), jnp.float32),
            pltpu.SemaphoreType.DMA,
            pltpu.SemaphoreType.DMA,
            pltpu.SemaphoreType.DMA,
            pltpu.SemaphoreType.DMA,
            pltpu.SemaphoreType.DMA,
            pltpu.SemaphoreType.DMA,
        ],
    )
    def k(idx_hbm, out_hbm, idx_v, cnt_v, os0, os1, is0, is1, is2, is3):
        wid = lax.axis_index("s") * nc + lax.axis_index("c")
        iota16 = lax.iota(jnp.int32, 16)
        ones = jnp.ones((16,), jnp.float32)
        zeros = jnp.zeros((16,), jnp.float32)
        osems = (os0, os1)
        isems = (is0, is1, is2, is3)

        def idx_fetch(g, s, start):
            @pl.when(g < GROUPS)
            def _():
                cp = (pltpu.async_copy if start else pltpu.make_async_copy)(
                    idx_hbm.at[pl.ds(g * 16, 16)], idx_v.at[s], isems[s])
                if not start:
                    cp.wait()

        idx_fetch(wid, 0, True)

        def zbody(j, _):
            for b in range(2):
                for r in range(16):
                    cnt_v[b, r, pl.ds(j * 16, 16)] = zeros
            return 0
        lax.fori_loop(0, VOCABP // 16, zbody, 0)

        def scatter_pass(s, b, val):
            def tbody(t, _):
                tvec = jnp.full((16,), t, jnp.int32)
                tok = plsc.load_gather(idx_v.at[s], [iota16, tvec])
                plsc.addupdate_scatter(cnt_v.at[b], [iota16, tok], val)
                return 0
            lax.fori_loop(0, TOK, tbody, 0, unroll=8)

        def swap_pass(s_old, s_new, b):
            def tbody(t, _):
                tvec = jnp.full((16,), t, jnp.int32)
                tok_o = plsc.load_gather(idx_v.at[s_old], [iota16, tvec])
                tok_n = plsc.load_gather(idx_v.at[s_new], [iota16, tvec])
                plsc.addupdate_scatter(cnt_v.at[b], [iota16, tok_o], -ones)
                plsc.addupdate_scatter(cnt_v.at[b], [iota16, tok_n], ones)
                return 0
            lax.fori_loop(0, TOK, tbody, 0, unroll=4)

        nblk = -(-gpt // 4)

        def block(i, _):
            for u in range(4):
                jj = i * 4 + u
                g = jj * nw + wid
                b = u % 2
                idx_fetch(g + nw, (u + 1) % 4, True)
                idx_fetch(g, u, False)

                @pl.when((g < GROUPS) & (jj >= 2))
                def _():
                    gprev = g - 2 * nw
                    pltpu.make_async_copy(
                        cnt_v.at[b], out_hbm.at[pl.ds(gprev * 16, 16)], osems[b]
                    ).wait()
                    swap_pass((u + 2) % 4, u, b)

                @pl.when((g < GROUPS) & (jj < 2))
                def _():
                    scatter_pass(u, b, ones)

                @pl.when(g < GROUPS)
                def _():
                    pltpu.async_copy(
                        cnt_v.at[b], out_hbm.at[pl.ds(g * 16, 16)], osems[b]
                    )
            return 0

        lax.fori_loop(0, nblk, block, 0)

        for b in range(2):
            pltpu.make_async_copy(
                cnt_v.at[b], out_hbm.at[pl.ds(0, 16)], osems[b]
            ).wait()

    return k(idx)


def _softmax_row(x):
    m = jnp.max(x, axis=1, keepdims=True)
    e = jnp.exp(x - m)
    return e / jnp.sum(e, axis=1, keepdims=True)


def _row_norms(enc):
    n = jnp.sqrt(jnp.sum(enc * enc, axis=1))
    return jnp.maximum(n, EPS)[None, :]


def _vnorm(q):
    return jnp.maximum(jnp.sqrt(jnp.sum(q * q)), EPS)


def _dense_body(cnt_ref, e_ref, ce_ref, r_ref, r2_ref, out_ref):
    E = e_ref[...]
    f32 = jnp.float32
    enc_k = jnp.dot(cnt_ref[R_KEYS:R_KEYS + NKEY, :], E, preferred_element_type=f32)
    enc_v = jnp.dot(cnt_ref[R_VALS:R_VALS + NKEY, :], E, preferred_element_type=f32)
    enc_c = jnp.dot(cnt_ref[R_CAND:R_CAND + NCAND, :], ce_ref[...], preferred_element_type=f32)
    enc_p = jnp.dot(cnt_ref[R_PERS:R_PERS + NPER, :], E, preferred_element_type=f32)
    q = jnp.dot(cnt_ref[R_XS:R_XS + 1, :], E, preferred_element_type=f32)

    nk = _row_norms(enc_k)
    nc = _row_norms(enc_c)
    np_ = _row_norms(enc_p)

    def dotq(v, enc):
        return lax.dot_general(v, enc, (((1,), (1,)), ((), ())),
                               preferred_element_type=f32)

    for rm_ref in (r_ref, r2_ref):
        Rm = rm_ref[...]
        cos = dotq(q, enc_p) / (np_ * _vnorm(q))
        ret = _softmax_row(cos)
        hop = jnp.dot(ret, enc_p, preferred_element_type=f32)
        q_plus = dotq(q + hop, Rm)
        cos2 = dotq(q_plus, enc_k) / (nk * _vnorm(q_plus))
        ret2 = _softmax_row(cos2)
        hop2 = jnp.dot(ret2, enc_v, preferred_element_type=f32)
        q = dotq(q_plus + hop2, Rm)

    out_ref[...] = dotq(q, enc_c) / (nc * _vnorm(q))


def _dense(counts, epad, cepad, R, R2):
    return pl.pallas_call(
        _dense_body,
        out_shape=jax.ShapeDtypeStruct((1, NCAND), jnp.float32),
    )(counts, epad, cepad, R, R2)


def kernel(xs, candidates, persona, keys, values, label, shared_emb, cand_emb, R, R2):
    del label
    i32 = jnp.int32
    xs_pad = jnp.pad(xs.astype(i32), ((0, 0), (0, TOK - xs.shape[1])),
                     constant_values=SENTINEL)
    idx = jnp.concatenate([
        keys.astype(i32), values.astype(i32), candidates.astype(i32),
        persona.astype(i32), xs_pad,
    ], axis=0)
    idx = jnp.pad(idx, ((0, ROWSP - ROWS), (0, 0)), constant_values=SENTINEL)

    epad = jnp.pad(shared_emb, ((0, VOCABP - shared_emb.shape[0]), (0, 0)))
    cepad = jnp.pad(cand_emb, ((0, VOCABP - cand_emb.shape[0]), (0, 0)))

    counts = _build_counts_sc(idx)
    preds = _dense(counts, epad, cepad, R, R2)
    return preds.reshape(NCAND)
